# baseline (speedup 1.0000x reference)
import jax
import jax.numpy as jnp
from jax.experimental import pallas as pl
from jax.experimental.pallas import tpu as pltpu

B = 128
V = 100000
NC = 782
NCM = 781
NG = 7
WIN = 962
NWIN = 13
NT = V // 8
RB = 32
NEG_INF = float("-inf")


def _max_body(x_ref, m_ref, macc):
    w = pl.program_id(0)

    @pl.when(w == 0)
    def _():
        macc[...] = jnp.full((8, 128), NEG_INF, jnp.float32)

    valid = 12500 - WIN * w

    def step(t, acc):
        sl = x_ref[pl.ds(t * 8, 8), :]
        sl = jnp.where(t < valid, sl, NEG_INF)
        return jnp.maximum(acc, sl)

    macc[...] = jax.lax.fori_loop(0, WIN, step, macc[...])

    @pl.when(w == NWIN - 1)
    def _():
        m_ref[...] = jnp.max(macc[...], axis=0, keepdims=True)


def _z_body(x_ref, m_ref, z_ref, tot):
    w = pl.program_id(0)

    @pl.when(w == 0)
    def _():
        tot[...] = jnp.zeros((1, 128), jnp.float32)

    m = m_ref[...]
    valid = 12500 - WIN * w

    def step(t, acc):
        sl = x_ref[pl.ds(t * 8, 8), :]
        e = jnp.exp(sl - m)
        e = jnp.where(t < valid, e, jnp.float32(0.0))
        return acc + e

    acc = jax.lax.fori_loop(0, WIN, step, jnp.zeros((8, 128), jnp.float32))
    b4 = acc[0:4, :] + acc[4:8, :]
    b2 = b4[0:2, :] + b4[2:4, :]
    b1 = b2[0:1, :] + b2[1:2, :]
    tot[...] = tot[...] + b1

    @pl.when(w == NWIN - 1)
    def _():
        z_ref[...] = tot[...]


def _t_body(x_ref, m_ref, z_ref, t_ref):
    m = m_ref[...]
    z = z_ref[...]

    def step(t, acc):
        for u in range(8):
            e = jnp.exp(x_ref[t * 8 + u] - m)
            acc = acc + e / z
        return acc

    t_ref[:, 0:NCM] = jax.lax.fori_loop(0, 16, step,
                                        jnp.zeros((RB, NCM), jnp.float32))
    t_ref[:, NCM:NC] = jnp.zeros((RB, 1), jnp.float32)


def _off_body(t_ref, out_ref, u_scr):
    def step(h, acc):
        acc = acc + t_ref[h]
        u_scr[h] = acc
        return acc

    g_tot = jax.lax.fori_loop(0, 128, step, jnp.zeros((NG, 128), jnp.float32))
    rows = []
    a = jnp.zeros((1, 128), jnp.float32)
    for g in range(NG):
        rows.append(a)
        a = a + g_tot[g:g + 1, :]
    off3 = jnp.concatenate(rows, axis=0)

    def step2(h, _):
        out_ref[h] = off3 + u_scr[h]
        return 0

    jax.lax.fori_loop(0, 128, step2, 0)


def _count_body(x_ref, xt_ref, m_ref, z_ref, off_ref, r_ref, a_ref, p_ref):
    m = m_ref[...]
    z = z_ref[...]
    off = off_ref[:, 0:NCM]
    off_t = off_ref[:, NCM:NC]
    r = r_ref[...]
    c_iota = jax.lax.broadcasted_iota(jnp.int32, (RB, NCM), 1)

    def step(t, carry):
        acc, cnt = carry
        for u in range(8):
            i = t * 8 + u
            e = jnp.exp(x_ref[i] - m)
            p = e / z
            acc = acc + p
            s = acc + off
            ok = s < r
            cnt = cnt + ok.astype(jnp.int32)
        return acc, cnt

    _, cnt = jax.lax.fori_loop(
        0, 16, step,
        (jnp.zeros((RB, NCM), jnp.float32), jnp.zeros((RB, NCM), jnp.int32)))

    acc_t = jnp.zeros((RB, 1), jnp.float32)
    cnt_t = jnp.zeros((RB, 1), jnp.int32)
    for u in range(32):
        i = 96 + u
        p = jnp.exp(xt_ref[i] - m) / z
        acc_t = acc_t + p
        s = acc_t + off_t
        cnt_t = cnt_t + (s < r).astype(jnp.int32)

    k = jnp.sum(cnt, axis=1, keepdims=True) + cnt_t
    a = jnp.minimum(k, V - 1)
    a_ref[...] = a
    c_star = a // 128
    i_star = a - c_star * 128
    c_hit = c_iota == c_star

    def step2(t, sel):
        for u in range(8):
            i = t * 8 + u
            hit = jnp.logical_and(c_hit, i == i_star)
            p = jnp.exp(x_ref[i] - m) / z
            sel = sel + jnp.where(hit, p, jnp.float32(0.0))
        return sel

    sel = jax.lax.fori_loop(0, 16, step2, jnp.zeros((RB, NCM), jnp.float32))
    tail_sel = c_star == NCM
    sel_t = jnp.zeros((RB, 1), jnp.float32)
    for u in range(32):
        i = 96 + u
        hit = jnp.logical_and(tail_sel, i == i_star + 96)
        p = jnp.exp(xt_ref[i] - m) / z
        sel_t = sel_t + jnp.where(hit, p, jnp.float32(0.0))
    p_ref[...] = jnp.sum(sel, axis=1, keepdims=True) + sel_t


def kernel(X, r):
    Xt = X.T
    Xr = jnp.transpose(X[:, :NCM * 128].reshape(B, NCM, 128), (2, 0, 1))
    Xtl = jnp.transpose(X[:, V - 128:].reshape(B, 1, 128), (2, 0, 1))

    m = pl.pallas_call(
        _max_body,
        grid=(NWIN,),
        in_specs=[pl.BlockSpec((WIN * 8, B), lambda w: (w, 0))],
        out_specs=pl.BlockSpec((1, B), lambda w: (0, 0)),
        out_shape=jax.ShapeDtypeStruct((1, B), jnp.float32),
        scratch_shapes=[pltpu.VMEM((8, 128), jnp.float32)],
    )(Xt)

    z = pl.pallas_call(
        _z_body,
        grid=(NWIN,),
        in_specs=[pl.BlockSpec((WIN * 8, B), lambda w: (w, 0)),
                  pl.BlockSpec((1, B), lambda w: (0, 0))],
        out_specs=pl.BlockSpec((1, B), lambda w: (0, 0)),
        out_shape=jax.ShapeDtypeStruct((1, B), jnp.float32),
        scratch_shapes=[pltpu.VMEM((1, 128), jnp.float32)],
    )(Xt, m)

    mb = m.reshape(B, 1)
    zb = z.reshape(B, 1)

    T = pl.pallas_call(
        _t_body,
        grid=(B // RB,),
        in_specs=[pl.BlockSpec((128, RB, NCM), lambda i: (0, i, 0)),
                  pl.BlockSpec((RB, 1), lambda i: (i, 0)),
                  pl.BlockSpec((RB, 1), lambda i: (i, 0))],
        out_specs=pl.BlockSpec((RB, NC), lambda i: (i, 0)),
        out_shape=jax.ShapeDtypeStruct((B, NC), jnp.float32),
    )(Xr, mb, zb)

    Tp = jnp.pad(T, ((0, 0), (0, NG * 128 - NC)))
    Tt = jnp.transpose(Tp.reshape(B, NG, 128), (2, 1, 0))

    Coff = pl.pallas_call(
        _off_body,
        grid=(1,),
        in_specs=[pl.BlockSpec((128, NG, B), lambda i: (0, 0, 0))],
        out_specs=pl.BlockSpec((128, NG, B), lambda i: (0, 0, 0)),
        out_shape=jax.ShapeDtypeStruct((128, NG, B), jnp.float32),
        scratch_shapes=[pltpu.VMEM((128, NG, 128), jnp.float32)],
    )(Tt)

    Coffr = jnp.transpose(Coff, (2, 1, 0)).reshape(B, NG * 128)[:, :NC]
    offset = jnp.concatenate(
        [jnp.zeros((B, 1), jnp.float32), Coffr[:, :NC - 1]], axis=1)

    a, probs = pl.pallas_call(
        _count_body,
        grid=(B // RB,),
        in_specs=[pl.BlockSpec((128, RB, NCM), lambda i: (0, i, 0)),
                  pl.BlockSpec((128, RB, 1), lambda i: (0, i, 0)),
                  pl.BlockSpec((RB, 1), lambda i: (i, 0)),
                  pl.BlockSpec((RB, 1), lambda i: (i, 0)),
                  pl.BlockSpec((RB, NC), lambda i: (i, 0)),
                  pl.BlockSpec((RB, 1), lambda i: (i, 0))],
        out_specs=[pl.BlockSpec((RB, 1), lambda i: (i, 0)),
                   pl.BlockSpec((RB, 1), lambda i: (i, 0))],
        out_shape=[jax.ShapeDtypeStruct((B, 1), jnp.int32),
                   jax.ShapeDtypeStruct((B, 1), jnp.float32)],
    )(Xr, Xtl, mb, zb, offset, r.reshape(B, 1))

    return a.reshape(B), probs.reshape(B)


# speedup vs baseline: 1.7673x; 1.7673x over previous
import jax
import jax.numpy as jnp
from jax.experimental import pallas as pl
from jax.experimental.pallas import tpu as pltpu

B = 128
V = 100000
NC = 782
NCM = 781
NG = 7
WIN = 962
NWIN = 13
NT = V // 8
RB = 32
NEG_INF = float("-inf")


def _max_body(x_ref, m_ref, macc):
    w = pl.program_id(0)

    @pl.when(w == 0)
    def _():
        macc[...] = jnp.full((8, 128), NEG_INF, jnp.float32)

    valid = 12500 - WIN * w

    def step(t, acc):
        for u in range(13):
            g = t * 13 + u
            sl = x_ref[pl.ds(g * 8, 8), :]
            sl = jnp.where(g < valid, sl, NEG_INF)
            acc = jnp.maximum(acc, sl)
        return acc

    macc[...] = jax.lax.fori_loop(0, WIN // 13, step, macc[...])

    @pl.when(w == NWIN - 1)
    def _():
        m_ref[...] = jnp.max(macc[...], axis=0, keepdims=True)


def _z_body(x_ref, m_ref, z_ref, tot):
    w = pl.program_id(0)

    @pl.when(w == 0)
    def _():
        tot[...] = jnp.zeros((1, 128), jnp.float32)

    m = m_ref[...]
    valid = 12500 - WIN * w

    def step(t, acc):
        for u in range(13):
            g = t * 13 + u
            sl = x_ref[pl.ds(g * 8, 8), :]
            e = jnp.exp(sl - m)
            e = jnp.where(g < valid, e, jnp.float32(0.0))
            acc = acc + e
        return acc

    acc = jax.lax.fori_loop(0, WIN // 13, step,
                            jnp.zeros((8, 128), jnp.float32))
    b4 = acc[0:4, :] + acc[4:8, :]
    b2 = b4[0:2, :] + b4[2:4, :]
    b1 = b2[0:1, :] + b2[1:2, :]
    tot[...] = tot[...] + b1

    @pl.when(w == NWIN - 1)
    def _():
        z_ref[...] = tot[...]


def _t_body(x_ref, m_ref, z_ref, t_ref):
    m = m_ref[...]
    z = z_ref[...]

    def step(t, acc):
        for u in range(8):
            e = jnp.exp(x_ref[t * 8 + u] - m)
            acc = acc + e / z
        return acc

    t_ref[...] = jax.lax.fori_loop(0, 16, step,
                                   jnp.zeros((RB, NC), jnp.float32))


def _off_body(t_ref, out_ref, u_scr):
    def step(h, acc):
        acc = acc + t_ref[h]
        u_scr[h] = acc
        return acc

    g_tot = jax.lax.fori_loop(0, 128, step, jnp.zeros((NG, 128), jnp.float32))
    rows = []
    a = jnp.zeros((1, 128), jnp.float32)
    for g in range(NG):
        rows.append(a)
        a = a + g_tot[g:g + 1, :]
    off3 = jnp.concatenate(rows, axis=0)

    def step2(h, _):
        out_ref[h] = off3 + u_scr[h]
        return 0

    jax.lax.fori_loop(0, 128, step2, 0)


def _count_body(x_ref, m_ref, z_ref, off_ref, r_ref, a_ref, p_ref):
    m = m_ref[...]
    z = z_ref[...]
    off = off_ref[...]
    r = r_ref[...]
    c_iota = jax.lax.broadcasted_iota(jnp.int32, (RB, NC), 1)
    tail_ok = c_iota != (NC - 1)

    def step(t, carry):
        acc, cnt = carry
        for u in range(8):
            i = t * 8 + u
            e = jnp.exp(x_ref[i] - m)
            p = e / z
            acc = acc + p
            s = acc + off
            valid = jnp.logical_or(i < 32, tail_ok)
            ok = jnp.logical_and(s < r, valid)
            cnt = cnt + ok.astype(jnp.int32)
        return acc, cnt

    _, cnt = jax.lax.fori_loop(
        0, 16, step,
        (jnp.zeros((RB, NC), jnp.float32), jnp.zeros((RB, NC), jnp.int32)))
    k = jnp.sum(cnt, axis=1, keepdims=True)
    a = jnp.minimum(k, V - 1)
    a_ref[...] = a
    c_star = a // 128
    i_star = a - c_star * 128
    c_hit = c_iota == c_star

    def step2(t, sel):
        for u in range(8):
            i = t * 8 + u
            hit = jnp.logical_and(c_hit, i == i_star)
            p = jnp.exp(x_ref[i] - m) / z
            sel = sel + jnp.where(hit, p, jnp.float32(0.0))
        return sel

    sel = jax.lax.fori_loop(0, 16, step2, jnp.zeros((RB, NC), jnp.float32))
    p_ref[...] = jnp.sum(sel, axis=1, keepdims=True)


def kernel(X, r):
    Xt = X.T
    Xp = jnp.pad(X, ((0, 0), (0, NC * 128 - V)), constant_values=-jnp.inf)
    Xr = jnp.transpose(Xp.reshape(B, NC, 128), (2, 0, 1))

    m = pl.pallas_call(
        _max_body,
        grid=(NWIN,),
        in_specs=[pl.BlockSpec((WIN * 8, B), lambda w: (w, 0))],
        out_specs=pl.BlockSpec((1, B), lambda w: (0, 0)),
        out_shape=jax.ShapeDtypeStruct((1, B), jnp.float32),
        scratch_shapes=[pltpu.VMEM((8, 128), jnp.float32)],
    )(Xt)

    z = pl.pallas_call(
        _z_body,
        grid=(NWIN,),
        in_specs=[pl.BlockSpec((WIN * 8, B), lambda w: (w, 0)),
                  pl.BlockSpec((1, B), lambda w: (0, 0))],
        out_specs=pl.BlockSpec((1, B), lambda w: (0, 0)),
        out_shape=jax.ShapeDtypeStruct((1, B), jnp.float32),
        scratch_shapes=[pltpu.VMEM((1, 128), jnp.float32)],
    )(Xt, m)

    mb = m.reshape(B, 1)
    zb = z.reshape(B, 1)

    T = pl.pallas_call(
        _t_body,
        grid=(B // RB,),
        in_specs=[pl.BlockSpec((128, RB, NC), lambda i: (0, i, 0)),
                  pl.BlockSpec((RB, 1), lambda i: (i, 0)),
                  pl.BlockSpec((RB, 1), lambda i: (i, 0))],
        out_specs=pl.BlockSpec((RB, NC), lambda i: (i, 0)),
        out_shape=jax.ShapeDtypeStruct((B, NC), jnp.float32),
    )(Xr, mb, zb)

    Tp = jnp.pad(T, ((0, 0), (0, NG * 128 - NC)))
    Tt = jnp.transpose(Tp.reshape(B, NG, 128), (2, 1, 0))

    Coff = pl.pallas_call(
        _off_body,
        grid=(1,),
        in_specs=[pl.BlockSpec((128, NG, B), lambda i: (0, 0, 0))],
        out_specs=pl.BlockSpec((128, NG, B), lambda i: (0, 0, 0)),
        out_shape=jax.ShapeDtypeStruct((128, NG, B), jnp.float32),
        scratch_shapes=[pltpu.VMEM((128, NG, 128), jnp.float32)],
    )(Tt)

    Coffr = jnp.transpose(Coff, (2, 1, 0)).reshape(B, NG * 128)[:, :NC]
    offset = jnp.concatenate(
        [jnp.zeros((B, 1), jnp.float32), Coffr[:, :NC - 1]], axis=1)

    a, probs = pl.pallas_call(
        _count_body,
        grid=(B // RB,),
        in_specs=[pl.BlockSpec((128, RB, NC), lambda i: (0, i, 0)),
                  pl.BlockSpec((RB, 1), lambda i: (i, 0)),
                  pl.BlockSpec((RB, 1), lambda i: (i, 0)),
                  pl.BlockSpec((RB, NC), lambda i: (i, 0)),
                  pl.BlockSpec((RB, 1), lambda i: (i, 0))],
        out_specs=[pl.BlockSpec((RB, 1), lambda i: (i, 0)),
                   pl.BlockSpec((RB, 1), lambda i: (i, 0))],
        out_shape=[jax.ShapeDtypeStruct((B, 1), jnp.int32),
                   jax.ShapeDtypeStruct((B, 1), jnp.float32)],
    )(Xr, mb, zb, offset, r.reshape(B, 1))

    return a.reshape(B), probs.reshape(B)
